# baseline (speedup 1.0000x reference)
import functools

import jax
import jax.numpy as jnp
from jax import lax
from jax.experimental import pallas as pl
from jax.experimental.pallas import tpu as pltpu
from jax.experimental.pallas import tpu_sc as plsc

N_NODES = 10000
N_EDGES = 320000
D = 128

NC = 2
NS = 16
NW = NC * NS
EPT = N_EDGES // NW
CH = 80
NCH = EPT // CH
N_PAD = 10240
RPT = N_PAD // NS
ZROWS = 128
NBUF = 8

_sc_mesh = plsc.VectorSubcoreMesh(core_axis_name="c", subcore_axis_name="s")


def _sage_agg_body(with_hist, xb_hbm, src_hbm, dst_hbm, *refs):
    if with_hist:
        (agg_out, hist_out, src_v, dst_v, rows_v, zbuf_v, hist_v,
         acc_sh, sem) = refs
    else:
        agg_out, src_v, dst_v, rows_v, zbuf_v, acc_sh, sem = refs
        hist_out = hist_v = None
    c = lax.axis_index("c")
    s = lax.axis_index("s")
    wid = s * NC + c

    pltpu.sync_copy(src_hbm.at[wid], src_v)
    pltpu.sync_copy(dst_hbm.at[wid], dst_v)

    zeros32 = jnp.zeros((32,), jnp.bfloat16)

    def zrow(i, carry):
        for k in range(D // 32):
            zbuf_v[i, pl.ds(k * 32, 32)] = zeros32
        return carry

    lax.fori_loop(0, ZROWS, zrow, 0)
    if with_hist:
        zeros16 = jnp.zeros((16,), jnp.float32)

        def zhist(i, carry):
            hist_v[pl.ds(i * 16, 16)] = zeros16
            return carry

        lax.fori_loop(0, N_PAD // 16, zhist, 0)

    for r in range(RPT // ZROWS):
        pltpu.sync_copy(zbuf_v, acc_sh.at[pl.ds(s * RPT + r * ZROWS, ZROWS)])
    plsc.subcore_barrier()

    ones16 = jnp.ones((16,), jnp.float32)

    for b in range(NBUF):
        pltpu.async_copy(xb_hbm.at[src_v.at[b]], rows_v.at[b], sem)

    def step(j, carry):
        b = lax.rem(j, NBUF)
        pltpu.make_async_copy(xb_hbm.at[src_v.at[j]],
                              rows_v.at[b], sem).wait()
        pltpu.sync_copy(rows_v.at[b], acc_sh.at[dst_v.at[j]], add=True)
        if with_hist:
            for k in range(CH // 16):
                idx = dst_v[j, pl.ds(k * 16, 16)]
                plsc.addupdate_scatter(hist_v, [idx], ones16)

        @pl.when(j + NBUF < NCH)
        def _():
            pltpu.async_copy(xb_hbm.at[src_v.at[j + NBUF]], rows_v.at[b], sem)

        return carry

    lax.fori_loop(0, NCH, step, 0)
    plsc.subcore_barrier()

    pltpu.sync_copy(acc_sh.at[pl.ds(s * RPT, RPT)],
                    agg_out.at[c, pl.ds(s * RPT, RPT)])
    if with_hist:
        pltpu.sync_copy(hist_v, hist_out.at[wid])


def _make_sc_agg(with_hist):
    out_type = [jax.ShapeDtypeStruct((NC, N_PAD, D), jnp.bfloat16)]
    scratch = [
        pltpu.VMEM((NCH, CH), jnp.int32),
        pltpu.VMEM((NCH, CH), jnp.int32),
        pltpu.VMEM((NBUF, CH, D), jnp.bfloat16),
        pltpu.VMEM((ZROWS, D), jnp.bfloat16),
    ]
    if with_hist:
        out_type.append(jax.ShapeDtypeStruct((NW, N_PAD), jnp.float32))
        scratch.append(pltpu.VMEM((N_PAD,), jnp.float32))
    scratch.append(pltpu.VMEM_SHARED((N_PAD, D), jnp.bfloat16))
    scratch.append(pltpu.SemaphoreType.DMA)
    return pl.kernel(
        functools.partial(_sage_agg_body, with_hist),
        out_type=out_type,
        mesh=_sc_mesh,
        scratch_types=scratch,
        compiler_params=pltpu.CompilerParams(needs_layout_passes=False,
                                             use_tc_tiling_on_sc=False,
                                             skip_device_barrier=True),
    )


_sc_agg_hist = _make_sc_agg(True)
_sc_agg = _make_sc_agg(False)


def _dense1_body(x_ref, agg_ref, hist_ref, wl_ref, wr_ref, b_ref, hb_ref):
    cnt = jnp.sum(hist_ref[...], axis=0)
    inv = 1.0 / jnp.maximum(cnt, 1.0)
    agg = (agg_ref[0].astype(jnp.float32) + agg_ref[1].astype(jnp.float32))
    mean = agg * inv[:, None]
    h = (jnp.dot(mean, wl_ref[...], preferred_element_type=jnp.float32)
         + b_ref[...]
         + jnp.dot(x_ref[...], wr_ref[...], preferred_element_type=jnp.float32))
    hb_ref[...] = jnp.maximum(h, 0.0).astype(jnp.bfloat16)


def _dense2_body(h_ref, agg_ref, hist_ref, wl_ref, wr_ref, b_ref, w3_ref,
                 b3_ref, o_ref):
    cnt = jnp.sum(hist_ref[...], axis=0)
    inv = 1.0 / jnp.maximum(cnt, 1.0)
    agg = (agg_ref[0].astype(jnp.float32) + agg_ref[1].astype(jnp.float32))
    mean = agg * inv[:, None]
    h2 = (jnp.dot(mean, wl_ref[...], preferred_element_type=jnp.float32)
          + b_ref[...]
          + jnp.dot(h_ref[...], wr_ref[...], preferred_element_type=jnp.float32))
    h2 = jnp.maximum(h2, 0.0)
    o_ref[...] = jnp.sum(h2 * w3_ref[...], axis=1, keepdims=True) + b3_ref[0, 0]


_R = 1024
_GRID = N_PAD // _R

_row_spec = pl.BlockSpec((_R, D), lambda i: (i, 0))
_agg_spec = pl.BlockSpec((NC, _R, D), lambda i: (0, i, 0))
_hist_spec = pl.BlockSpec((NW, _R), lambda i: (0, i))
_full = pl.BlockSpec((D, D), lambda i: (0, 0))
_bias_spec = pl.BlockSpec((1, D), lambda i: (0, 0))

_dense1 = pl.pallas_call(
    _dense1_body,
    grid=(_GRID,),
    in_specs=[_row_spec, _agg_spec, _hist_spec, _full, _full, _bias_spec],
    out_specs=_row_spec,
    out_shape=jax.ShapeDtypeStruct((N_NODES, D), jnp.bfloat16),
)

_dense2 = pl.pallas_call(
    _dense2_body,
    grid=(_GRID,),
    in_specs=[_row_spec, _agg_spec, _hist_spec, _full, _full, _bias_spec,
              _bias_spec, pl.BlockSpec((1, 1), lambda i: (0, 0))],
    out_specs=pl.BlockSpec((_R, 1), lambda i: (i, 0)),
    out_shape=jax.ShapeDtypeStruct((N_NODES, 1), jnp.float32),
)


def kernel(x, edge_index, W1_l, W1_r, b1, W2_l, W2_r, b2, W3, b3):
    src = edge_index[0].astype(jnp.int32).reshape(NW, NCH, CH)
    dst = edge_index[1].astype(jnp.int32).reshape(NW, NCH, CH)
    xb = x.astype(jnp.bfloat16)

    agg1, hist = _sc_agg_hist(xb, src, dst)
    hb = _dense1(x, agg1, hist, W1_l.T, W1_r.T, b1.reshape(1, D))
    (agg2,) = _sc_agg(hb, src, dst)
    out = _dense2(hb, agg2, hist, W2_l.T, W2_r.T, b2.reshape(1, D),
                  W3, b3.reshape(1, 1))
    return out


# speedup vs baseline: 1.0041x; 1.0041x over previous
import functools

import jax
import jax.numpy as jnp
from jax import lax
from jax.experimental import pallas as pl
from jax.experimental.pallas import tpu as pltpu
from jax.experimental.pallas import tpu_sc as plsc

N_NODES = 10000
N_EDGES = 320000
D = 128

NC = 2
NS = 16
NW = NC * NS
EPT = N_EDGES // NW
CH = 80
NCH = EPT // CH
N_PAD = 10240
RPT = N_PAD // NS
ZROWS = 128
NBUF = 8

_sc_mesh = plsc.VectorSubcoreMesh(core_axis_name="c", subcore_axis_name="s")


def _sage_agg_body(with_hist, xb_hbm, src_hbm, dst_hbm, *refs):
    if with_hist:
        (agg_out, hist_out, src_v, dst_v, rows_v, zbuf_v, hist_v,
         acc_sh, sem, sem2) = refs
    else:
        agg_out, src_v, dst_v, rows_v, zbuf_v, acc_sh, sem, sem2 = refs
        hist_out = hist_v = None
    c = lax.axis_index("c")
    s = lax.axis_index("s")
    wid = s * NC + c

    pltpu.sync_copy(src_hbm.at[wid], src_v)
    pltpu.sync_copy(dst_hbm.at[wid], dst_v)

    zeros32 = jnp.zeros((32,), jnp.bfloat16)

    def zrow(i, carry):
        for k in range(D // 32):
            zbuf_v[i, pl.ds(k * 32, 32)] = zeros32
        return carry

    lax.fori_loop(0, ZROWS, zrow, 0)
    if with_hist:
        zeros16 = jnp.zeros((16,), jnp.float32)

        def zhist(i, carry):
            hist_v[pl.ds(i * 16, 16)] = zeros16
            return carry

        lax.fori_loop(0, N_PAD // 16, zhist, 0)

    for r in range(RPT // ZROWS):
        pltpu.sync_copy(zbuf_v, acc_sh.at[pl.ds(s * RPT + r * ZROWS, ZROWS)])
    plsc.subcore_barrier()

    ones16 = jnp.ones((16,), jnp.float32)

    for b in range(NBUF):
        pltpu.async_copy(xb_hbm.at[src_v.at[b]], rows_v.at[b], sem)

    def step(j, carry):
        b = lax.rem(j, NBUF)
        pltpu.make_async_copy(xb_hbm.at[src_v.at[j]],
                              rows_v.at[b], sem).wait()
        pltpu.async_copy(rows_v.at[b], acc_sh.at[dst_v.at[j]], sem2, add=True)
        if with_hist:
            for k in range(CH // 16):
                idx = dst_v[j, pl.ds(k * 16, 16)]
                plsc.addupdate_scatter(hist_v, [idx], ones16)

        bp = lax.rem(j + NBUF - 1, NBUF)

        @pl.when(j >= 1)
        def _():
            pltpu.make_async_copy(rows_v.at[bp],
                                  acc_sh.at[dst_v.at[j - 1]], sem2).wait()

            @pl.when(j - 1 + NBUF < NCH)
            def _():
                pltpu.async_copy(xb_hbm.at[src_v.at[j - 1 + NBUF]],
                                 rows_v.at[bp], sem)

        return carry

    lax.fori_loop(0, NCH, step, 0)
    pltpu.make_async_copy(rows_v.at[lax.rem(NCH - 1, NBUF)],
                          acc_sh.at[dst_v.at[NCH - 1]], sem2).wait()
    plsc.subcore_barrier()

    pltpu.sync_copy(acc_sh.at[pl.ds(s * RPT, RPT)],
                    agg_out.at[c, pl.ds(s * RPT, RPT)])
    if with_hist:
        pltpu.sync_copy(hist_v, hist_out.at[wid])


def _make_sc_agg(with_hist):
    out_type = [jax.ShapeDtypeStruct((NC, N_PAD, D), jnp.bfloat16)]
    scratch = [
        pltpu.VMEM((NCH, CH), jnp.int32),
        pltpu.VMEM((NCH, CH), jnp.int32),
        pltpu.VMEM((NBUF, CH, D), jnp.bfloat16),
        pltpu.VMEM((ZROWS, D), jnp.bfloat16),
    ]
    if with_hist:
        out_type.append(jax.ShapeDtypeStruct((NW, N_PAD), jnp.float32))
        scratch.append(pltpu.VMEM((N_PAD,), jnp.float32))
    scratch.append(pltpu.VMEM_SHARED((N_PAD, D), jnp.bfloat16))
    scratch.append(pltpu.SemaphoreType.DMA)
    scratch.append(pltpu.SemaphoreType.DMA)
    return pl.kernel(
        functools.partial(_sage_agg_body, with_hist),
        out_type=out_type,
        mesh=_sc_mesh,
        scratch_types=scratch,
        compiler_params=pltpu.CompilerParams(needs_layout_passes=False,
                                             use_tc_tiling_on_sc=False),
    )


_sc_agg_hist = _make_sc_agg(True)
_sc_agg = _make_sc_agg(False)


def _dense1_body(x_ref, agg_ref, hist_ref, wl_ref, wr_ref, b_ref, hb_ref):
    cnt = jnp.sum(hist_ref[...], axis=0)
    inv = 1.0 / jnp.maximum(cnt, 1.0)
    agg = (agg_ref[0].astype(jnp.float32) + agg_ref[1].astype(jnp.float32))
    mean = agg * inv[:, None]
    h = (jnp.dot(mean, wl_ref[...], preferred_element_type=jnp.float32)
         + b_ref[...]
         + jnp.dot(x_ref[...], wr_ref[...], preferred_element_type=jnp.float32))
    hb_ref[...] = jnp.maximum(h, 0.0).astype(jnp.bfloat16)


def _dense2_body(h_ref, agg_ref, hist_ref, wl_ref, wr_ref, b_ref, w3_ref,
                 b3_ref, o_ref):
    cnt = jnp.sum(hist_ref[...], axis=0)
    inv = 1.0 / jnp.maximum(cnt, 1.0)
    agg = (agg_ref[0].astype(jnp.float32) + agg_ref[1].astype(jnp.float32))
    mean = agg * inv[:, None]
    h2 = (jnp.dot(mean, wl_ref[...], preferred_element_type=jnp.float32)
          + b_ref[...]
          + jnp.dot(h_ref[...], wr_ref[...], preferred_element_type=jnp.float32))
    h2 = jnp.maximum(h2, 0.0)
    o_ref[...] = jnp.sum(h2 * w3_ref[...], axis=1, keepdims=True) + b3_ref[0, 0]


_R = 1024
_GRID = N_PAD // _R

_row_spec = pl.BlockSpec((_R, D), lambda i: (i, 0))
_agg_spec = pl.BlockSpec((NC, _R, D), lambda i: (0, i, 0))
_hist_spec = pl.BlockSpec((NW, _R), lambda i: (0, i))
_full = pl.BlockSpec((D, D), lambda i: (0, 0))
_bias_spec = pl.BlockSpec((1, D), lambda i: (0, 0))

_dense1 = pl.pallas_call(
    _dense1_body,
    grid=(_GRID,),
    in_specs=[_row_spec, _agg_spec, _hist_spec, _full, _full, _bias_spec],
    out_specs=_row_spec,
    out_shape=jax.ShapeDtypeStruct((N_NODES, D), jnp.bfloat16),
)

_dense2 = pl.pallas_call(
    _dense2_body,
    grid=(_GRID,),
    in_specs=[_row_spec, _agg_spec, _hist_spec, _full, _full, _bias_spec,
              _bias_spec, pl.BlockSpec((1, 1), lambda i: (0, 0))],
    out_specs=pl.BlockSpec((_R, 1), lambda i: (i, 0)),
    out_shape=jax.ShapeDtypeStruct((N_NODES, 1), jnp.float32),
)


def kernel(x, edge_index, W1_l, W1_r, b1, W2_l, W2_r, b2, W3, b3):
    src = edge_index[0].astype(jnp.int32).reshape(NW, NCH, CH)
    dst = edge_index[1].astype(jnp.int32).reshape(NW, NCH, CH)
    xb = x.astype(jnp.bfloat16)

    agg1, hist = _sc_agg_hist(xb, src, dst)
    hb = _dense1(x, agg1, hist, W1_l.T, W1_r.T, b1.reshape(1, D))
    (agg2,) = _sc_agg(hb, src, dst)
    out = _dense2(hb, agg2, hist, W2_l.T, W2_r.T, b2.reshape(1, D),
                  W3, b3.reshape(1, 1))
    return out
